# baseline (speedup 1.0000x reference)
import functools

import jax
import jax.numpy as jnp
from jax.experimental import pallas as pl
from jax.experimental.pallas import tpu as pltpu

N = 10000
E = 160000
D = 256
H = 2
C = 128
ESL = E + N
NPAD = 10240


def _feat_kernel(x_ref, w_ref, asrc_ref, adst_ref, xwh_ref, a4_ref):
    h = pl.program_id(0)
    xwb = jnp.dot(x_ref[...], w_ref[...], preferred_element_type=jnp.float32)
    att = jnp.concatenate(
        [asrc_ref[pl.ds(h, 1), :], adst_ref[pl.ds(h, 1), :]], axis=0)
    a = jax.lax.dot_general(xwb, att, (((1,), (1,)), ((), ())),
                            preferred_element_type=jnp.float32)
    xwh_ref[...] = xwb[None]
    a4_ref[...] = a[None]


def _compute_features(x, W, att_src, att_dst):
    BN = 1000
    grid = (H, N // BN)
    return pl.pallas_call(
        _feat_kernel,
        grid=grid,
        in_specs=[
            pl.BlockSpec((BN, D), lambda h, i: (i, 0)),
            pl.BlockSpec((D, C), lambda h, i: (0, h)),
            pl.BlockSpec((H, C), lambda h, i: (0, 0)),
            pl.BlockSpec((H, C), lambda h, i: (0, 0)),
        ],
        out_specs=[
            pl.BlockSpec((1, BN, C), lambda h, i: (h, i, 0)),
            pl.BlockSpec((1, BN, 2), lambda h, i: (h, i, 0)),
        ],
        out_shape=[
            jax.ShapeDtypeStruct((H, N, C), jnp.float32),
            jax.ShapeDtypeStruct((H, N, 2), jnp.float32),
        ],
    )(x, W, att_src, att_dst)


def _mlp_kernel(num_ref, den_ref, bc_ref, wa_ref, ba_ref, w1_ref, b1_ref,
                w2_ref, b2_ref, out_ref):
    den = den_ref[...]
    den = jnp.where(den > 0, den, 1.0)
    h0 = jnp.concatenate(
        [num_ref[0] / den[0][:, None], num_ref[1] / den[1][:, None]], axis=1)
    h0 = jax.nn.relu(h0 + bc_ref[...])
    h1 = jax.nn.relu(jnp.dot(h0, wa_ref[...], preferred_element_type=jnp.float32)
                     + ba_ref[...])
    h2 = jax.nn.relu(jnp.dot(h1, w1_ref[...], preferred_element_type=jnp.float32)
                     + b1_ref[...])
    h3 = jnp.dot(h2, w2_ref[...], preferred_element_type=jnp.float32) + b2_ref[...]
    out_ref[...] = h3


def _mlp(num, den, b_conv, Wa, ba, W1, b1, W2, b2):
    BN = 1024
    grid = (NPAD // BN,)
    return pl.pallas_call(
        _mlp_kernel,
        grid=grid,
        in_specs=[
            pl.BlockSpec((2, BN, C), lambda i: (0, i, 0)),
            pl.BlockSpec((2, BN), lambda i: (0, i)),
            pl.BlockSpec((1, H * C), lambda i: (0, 0)),
            pl.BlockSpec((H * C, 128), lambda i: (0, 0)),
            pl.BlockSpec((1, 128), lambda i: (0, 0)),
            pl.BlockSpec((128, 64), lambda i: (0, 0)),
            pl.BlockSpec((1, 64), lambda i: (0, 0)),
            pl.BlockSpec((64, 3), lambda i: (0, 0)),
            pl.BlockSpec((1, 3), lambda i: (0, 0)),
        ],
        out_specs=pl.BlockSpec((BN, 3), lambda i: (i, 0)),
        out_shape=jax.ShapeDtypeStruct((NPAD, 3), jnp.float32),
    )(num, den, b_conv.reshape(1, H * C), Wa, ba.reshape(1, 128),
      W1, b1.reshape(1, 64), W2, b2.reshape(1, 3))


def _cdist_kernel(hi_ref, hj_ref, out_ref):
    hi = hi_ref[...]
    hj = hj_ref[...]
    g = jax.lax.dot_general(hi, hj, (((1,), (1,)), ((), ())),
                            preferred_element_type=jnp.float32)
    sqi = jnp.sum(hi * hi, axis=1, keepdims=True)
    sqj = jnp.sum(hj * hj, axis=1)[None, :]
    d2 = sqi + sqj - 2.0 * g
    d2 = jnp.maximum(d2, 0.0)
    d = jnp.sqrt(jnp.where(d2 > 0, d2, 1.0))
    out_ref[...] = jnp.where(d2 > 0, d, 0.0)


def _cdist(h3):
    B = 512
    g = pl.cdiv(N, B)
    return pl.pallas_call(
        _cdist_kernel,
        grid=(g, g),
        in_specs=[
            pl.BlockSpec((B, 3), lambda i, j: (i, 0)),
            pl.BlockSpec((B, 3), lambda i, j: (j, 0)),
        ],
        out_specs=pl.BlockSpec((B, B), lambda i, j: (i, j)),
        out_shape=jax.ShapeDtypeStruct((N, N), jnp.float32),
    )(h3, h3)


def kernel(x, edge_index, W, att_src, att_dst, b_conv, Wa, ba, W1, b1, W2, b2):
    loop = jnp.arange(N, dtype=edge_index.dtype)
    src = jnp.concatenate([edge_index[0], loop])
    dst = jnp.concatenate([edge_index[1], loop])

    xwh, a4 = _compute_features(x, W, att_src, att_dst)

    a_src_nh = a4[:, :, 0].T
    a_dst_nh = a4[:, :, 1].T
    e = a_src_nh[src] + a_dst_nh[dst]
    e = jnp.where(e > 0, e, 0.2 * e)
    m = jax.ops.segment_max(e, dst, num_segments=N)
    m = jnp.where(jnp.isfinite(m), m, 0.0)
    ex = jnp.exp(e - m[dst])
    denom = jax.ops.segment_sum(ex, dst, num_segments=N)
    alpha = ex / (denom[dst] + 1e-16)
    xw3 = xwh.transpose(1, 0, 2)
    out = jax.ops.segment_sum(alpha[:, :, None] * xw3[src], dst, num_segments=N)
    num = jnp.pad(out.transpose(1, 0, 2), ((0, 0), (0, NPAD - N), (0, 0)))
    den = jnp.pad(jnp.ones((H, N), jnp.float32), ((0, 0), (0, NPAD - N)))

    h3 = _mlp(num, den, b_conv, Wa, ba, W1, b1, W2, b2)
    dist = _cdist(h3)
    edge_index_sl = jnp.stack([src, dst], axis=0)
    return dist, (edge_index_sl, alpha)


# speedup vs baseline: 15.0549x; 15.0549x over previous
import functools

import jax
import jax.numpy as jnp
from jax import lax
from jax.experimental import pallas as pl
from jax.experimental.pallas import tpu as pltpu
from jax.experimental.pallas import tpu_sc as plsc

N = 10000
E = 160000
D = 256
H = 2
C = 128
ESL = E + N
NPAD = 10240
NTILES = 16
TPT = 10752
EPAD = NTILES * TPT
NB = TPT // 128
NPT = NPAD // NTILES


def _feat_kernel(x_ref, w_ref, asrc_ref, adst_ref, xwh_ref, a4_ref):
    h = pl.program_id(0)
    xwb = jnp.dot(x_ref[...], w_ref[...], preferred_element_type=jnp.float32)
    att = jnp.concatenate(
        [asrc_ref[pl.ds(h, 1), :], adst_ref[pl.ds(h, 1), :]], axis=0)
    a = jax.lax.dot_general(xwb, att, (((1,), (1,)), ((), ())),
                            preferred_element_type=jnp.float32)
    xwh_ref[...] = xwb[None]
    a4_ref[...] = a[None]


def _compute_features(x, W, att_src, att_dst):
    BN = 1000
    grid = (H, N // BN)
    return pl.pallas_call(
        _feat_kernel,
        grid=grid,
        in_specs=[
            pl.BlockSpec((BN, D), lambda h, i: (i, 0)),
            pl.BlockSpec((D, C), lambda h, i: (0, h)),
            pl.BlockSpec((H, C), lambda h, i: (0, 0)),
            pl.BlockSpec((H, C), lambda h, i: (0, 0)),
        ],
        out_specs=[
            pl.BlockSpec((1, BN, C), lambda h, i: (h, i, 0)),
            pl.BlockSpec((1, BN, 2), lambda h, i: (h, i, 0)),
        ],
        out_shape=[
            jax.ShapeDtypeStruct((H, N, C), jnp.float32),
            jax.ShapeDtypeStruct((H, N, 2), jnp.float32),
        ],
    )(x, W, att_src, att_dst)


CW = 144


def _edge_body(src_hbm, dst_hbm, xwext_hbm, adst_hbm,
               num_hbm, alpha_hbm,
               srcv, dstv, gidx, didx, exb, rowsx, ddata,
               ex_all, alpha_b, zb, sem, num_s):
    c = lax.axis_index("c")
    s = lax.axis_index("s")
    base_e = s * TPT
    zero16 = jnp.zeros((16,), jnp.float32)
    iota16 = lax.iota(jnp.int32, 16)
    zidx = jnp.zeros((16,), jnp.int32)
    c128 = jnp.full((16,), 128, jnp.int32)
    unit = jnp.where(iota16 == 0, 1.0, 0.0).astype(jnp.float32)

    def _z1(i, _):
        for v in range(CW // 16):
            zb[i, pl.ds(v * 16, 16)] = zero16
        return 0
    lax.fori_loop(0, 32, _z1, 0)
    for k in range(NPT // 32):
        pltpu.sync_copy(zb, num_s.at[pl.ds(s * NPT + k * 32, 32)])
    plsc.subcore_barrier()

    def _batch1(b, _):
        off = base_e + b * 128
        pltpu.sync_copy(src_hbm.at[pl.ds(off, 128)], srcv)
        pltpu.sync_copy(dst_hbm.at[pl.ds(off, 128)], dstv)
        for v in range(8):
            sl = pl.ds(v * 16, 16)
            gidx[sl] = srcv[sl] + c * N
            didx[sl] = dstv[sl] + c * NPAD
        cp1 = pltpu.async_copy(xwext_hbm.at[gidx], rowsx, sem)
        cp1.wait()
        cp2 = pltpu.async_copy(adst_hbm.at[didx], ddata, sem)
        cp2.wait()
        for v in range(8):
            sl = pl.ds(v * 16, 16)
            rid = v * 16 + iota16
            asv = plsc.load_gather(rowsx, [rid, c128])
            adv = plsc.load_gather(ddata, [rid, zidx])
            e = asv + adv
            e = jnp.where(e > 0, e, 0.2 * e)
            exv = jnp.exp(e)
            ge = off + v * 16 + iota16
            exv = jnp.where(ge < ESL, exv, 0.0)
            exb[sl] = exv
            ex_all[pl.ds(b * 128 + v * 16, 16)] = exv
        def _scale(j, _):
            exj = exb[pl.ds(j, 16)][0]
            for v in range(8):
                sl = pl.ds(v * 16, 16)
                rowsx[j, sl] = rowsx[j, sl] * exj
            rowsx[j, pl.ds(128, 16)] = unit * exj
            return 0
        lax.fori_loop(0, 128, _scale, 0)
        pltpu.sync_copy(rowsx, num_s.at[dstv], add=True)
        return 0
    lax.fori_loop(0, NB, _batch1, 0)
    plsc.subcore_barrier()

    pltpu.sync_copy(num_s.at[pl.ds(s * NPT, NPT)],
                    num_hbm.at[pl.ds(c * NPAD + s * NPT, NPT)])
    plsc.subcore_barrier()

    def _batch2(b, _):
        off = base_e + b * 128
        pltpu.sync_copy(dst_hbm.at[pl.ds(off, 128)], dstv)
        for v in range(8):
            sl = pl.ds(v * 16, 16)
            didx[sl] = dstv[sl] + c * NPAD
        pltpu.async_copy(num_hbm.at[didx], rowsx, sem).wait()
        for v in range(8):
            sl = pl.ds(v * 16, 16)
            rid = v * 16 + iota16
            dv = plsc.load_gather(rowsx, [rid, c128])
            exv = ex_all[pl.ds(b * 128 + v * 16, 16)]
            alpha_b[sl] = exv / (dv + 1e-16)
        pltpu.sync_copy(alpha_b, alpha_hbm.at[c, pl.ds(off, 128)])
        return 0
    lax.fori_loop(0, NB, _batch2, 0)


def _edge_phase(src, dst, xwext, adst_tab):
    mesh = plsc.VectorSubcoreMesh(core_axis_name="c", subcore_axis_name="s")
    f = pl.kernel(
        _edge_body,
        out_type=[
            jax.ShapeDtypeStruct((H * NPAD, CW), jnp.float32),
            jax.ShapeDtypeStruct((H, EPAD), jnp.float32),
        ],
        mesh=mesh,
        compiler_params=pltpu.CompilerParams(
            needs_layout_passes=False, use_tc_tiling_on_sc=False),
        scratch_types=[
            pltpu.VMEM((128,), jnp.int32),
            pltpu.VMEM((128,), jnp.int32),
            pltpu.VMEM((128,), jnp.int32),
            pltpu.VMEM((128,), jnp.int32),
            pltpu.VMEM((144,), jnp.float32),
            pltpu.VMEM((128, CW), jnp.float32),
            pltpu.VMEM((128, 16), jnp.float32),
            pltpu.VMEM((TPT,), jnp.float32),
            pltpu.VMEM((128,), jnp.float32),
            pltpu.VMEM((32, CW), jnp.float32),
            pltpu.SemaphoreType.DMA,
            pltpu.VMEM_SHARED((NPAD, CW), jnp.float32),
        ],
    )
    return f(src, dst, xwext, adst_tab)


def _mlp_kernel(num_ref, bc_ref, wa_ref, ba_ref, w1_ref, b1_ref,
                w2_ref, b2_ref, out_ref):
    arr = num_ref[...]
    den = arr[:, :, 128]
    den = jnp.where(den > 0, den, 1.0)
    h0 = jnp.concatenate(
        [arr[0, :, :C] / den[0][:, None], arr[1, :, :C] / den[1][:, None]],
        axis=1)
    h0 = jax.nn.relu(h0 + bc_ref[...])
    h1 = jax.nn.relu(jnp.dot(h0, wa_ref[...], preferred_element_type=jnp.float32)
                     + ba_ref[...])
    h2 = jax.nn.relu(jnp.dot(h1, w1_ref[...], preferred_element_type=jnp.float32)
                     + b1_ref[...])
    h3 = jnp.dot(h2, w2_ref[...], preferred_element_type=jnp.float32) + b2_ref[...]
    out_ref[...] = h3


def _mlp(num, b_conv, Wa, ba, W1, b1, W2, b2):
    BN = 1024
    grid = (NPAD // BN,)
    return pl.pallas_call(
        _mlp_kernel,
        grid=grid,
        in_specs=[
            pl.BlockSpec((2, BN, CW), lambda i: (0, i, 0)),
            pl.BlockSpec((1, H * C), lambda i: (0, 0)),
            pl.BlockSpec((H * C, 128), lambda i: (0, 0)),
            pl.BlockSpec((1, 128), lambda i: (0, 0)),
            pl.BlockSpec((128, 64), lambda i: (0, 0)),
            pl.BlockSpec((1, 64), lambda i: (0, 0)),
            pl.BlockSpec((64, 3), lambda i: (0, 0)),
            pl.BlockSpec((1, 3), lambda i: (0, 0)),
        ],
        out_specs=pl.BlockSpec((BN, 3), lambda i: (i, 0)),
        out_shape=jax.ShapeDtypeStruct((NPAD, 3), jnp.float32),
    )(num, b_conv.reshape(1, H * C), Wa, ba.reshape(1, 128),
      W1, b1.reshape(1, 64), W2, b2.reshape(1, 3))


def _cdist_kernel(hi_ref, hj_ref, out_ref):
    hi = hi_ref[...]
    hj = hj_ref[...]
    g = jax.lax.dot_general(hi, hj, (((1,), (1,)), ((), ())),
                            preferred_element_type=jnp.float32)
    sqi = jnp.sum(hi * hi, axis=1, keepdims=True)
    sqj = jnp.sum(hj * hj, axis=1)[None, :]
    d2 = sqi + sqj - 2.0 * g
    d2 = jnp.maximum(d2, 0.0)
    d = jnp.sqrt(jnp.where(d2 > 0, d2, 1.0))
    out_ref[...] = jnp.where(d2 > 0, d, 0.0)


def _cdist(h3):
    B = 512
    g = pl.cdiv(N, B)
    return pl.pallas_call(
        _cdist_kernel,
        grid=(g, g),
        in_specs=[
            pl.BlockSpec((B, 3), lambda i, j: (i, 0)),
            pl.BlockSpec((B, 3), lambda i, j: (j, 0)),
        ],
        out_specs=pl.BlockSpec((B, B), lambda i, j: (i, j)),
        out_shape=jax.ShapeDtypeStruct((N, N), jnp.float32),
    )(h3, h3)


def kernel(x, edge_index, W, att_src, att_dst, b_conv, Wa, ba, W1, b1, W2, b2):
    loop = jnp.arange(N, dtype=edge_index.dtype)
    src = jnp.concatenate([edge_index[0], loop])
    dst = jnp.concatenate([edge_index[1], loop])

    xwh, a4 = _compute_features(x, W, att_src, att_dst)

    pad = jnp.zeros((EPAD - ESL,), src.dtype)
    src_p = jnp.concatenate([src, pad])
    dst_p = jnp.concatenate([dst, pad])
    xwflat = xwh.reshape(H * N, C)
    asrc_col = a4[:, :, 0].reshape(H * N, 1)
    xwext = jnp.concatenate(
        [xwflat, asrc_col, jnp.zeros((H * N, CW - C - 1), jnp.float32)], axis=1)
    adst_tab = jnp.pad(a4[:, :, 1], ((0, 0), (0, NPAD - N)))
    adst_tab = jnp.pad(adst_tab.reshape(H * NPAD, 1), ((0, 0), (0, 15)))
    numden, alpha_he = _edge_phase(src_p, dst_p, xwext, adst_tab)
    alpha = jnp.stack([alpha_he[0, :ESL], alpha_he[1, :ESL]], axis=1)

    num = numden.reshape(H, NPAD, CW)
    h3 = _mlp(num, b_conv, Wa, ba, W1, b1, W2, b2)
    dist = _cdist(h3)
    edge_index_sl = jnp.stack([src, dst], axis=0)
    return dist, (edge_index_sl, alpha)


# speedup vs baseline: 20.0584x; 1.3323x over previous
import functools

import jax
import jax.numpy as jnp
from jax import lax
from jax.experimental import pallas as pl
from jax.experimental.pallas import tpu as pltpu
from jax.experimental.pallas import tpu_sc as plsc

N = 10000
E = 160000
D = 256
H = 2
C = 128
ESL = E + N
NPAD = 10240
NTILES = 16
TPT = 10752
EPAD = NTILES * TPT
NB = TPT // 128
NPT = NPAD // NTILES


def _feat_kernel(x_ref, w_ref, asrc_ref, adst_ref, xwext_ref, adst16_ref):
    h = pl.program_id(0)
    bn = x_ref.shape[0]
    xwb = jnp.dot(x_ref[...], w_ref[...], preferred_element_type=jnp.float32)
    a_s = jax.lax.dot_general(xwb, asrc_ref[pl.ds(h, 1), :],
                              (((1,), (1,)), ((), ())),
                              preferred_element_type=jnp.float32)
    a_d = jax.lax.dot_general(xwb, adst_ref[pl.ds(h, 1), :],
                              (((1,), (1,)), ((), ())),
                              preferred_element_type=jnp.float32)
    zpad = jnp.zeros((bn, 15), jnp.float32)
    xwext_ref[...] = jnp.concatenate([xwb, a_s, zpad], axis=1)
    adst16_ref[...] = jnp.concatenate([a_d, zpad], axis=1)[None]


def _compute_features(x, W, att_src, att_dst):
    BN = 1000
    grid = (H, N // BN)
    nb = N // BN
    return pl.pallas_call(
        _feat_kernel,
        grid=grid,
        in_specs=[
            pl.BlockSpec((BN, D), lambda h, i: (i, 0)),
            pl.BlockSpec((D, C), lambda h, i: (0, h)),
            pl.BlockSpec((H, C), lambda h, i: (0, 0)),
            pl.BlockSpec((H, C), lambda h, i: (0, 0)),
        ],
        out_specs=[
            pl.BlockSpec((BN, CW), lambda h, i: (h * nb + i, 0)),
            pl.BlockSpec((1, BN, 16), lambda h, i: (h, i, 0)),
        ],
        out_shape=[
            jax.ShapeDtypeStruct((H * N, CW), jnp.float32),
            jax.ShapeDtypeStruct((H, NPAD, 16), jnp.float32),
        ],
    )(x, W, att_src, att_dst)


CW = 144
EB = 64
NBAT = TPT // EB


def _edge_body(src_hbm, dst_hbm, xwext_hbm, adst_hbm,
               num_hbm, alpha_hbm,
               srcv0, srcv1, dstv0, dstv1, gidx0, gidx1, didx0, didx1,
               sidx0, sidx1, exb0, exb1, rowsx0, rowsx1, ddata0, ddata1,
               ex_all, ab0, ab1, zb,
               semA0, semA1, semG0, semG1, semS0, semS1, num_s):
    srcv = (srcv0, srcv1)
    dstv = (dstv0, dstv1)
    gidx = (gidx0, gidx1)
    didx = (didx0, didx1)
    sidx = (sidx0, sidx1)
    exb = (exb0, exb1)
    rowsx = (rowsx0, rowsx1)
    ddata = (ddata0, ddata1)
    alpha_b = (ab0, ab1)
    semA = (semA0, semA1)
    semG = (semG0, semG1)
    semS = (semS0, semS1)

    c = lax.axis_index("c")
    s = lax.axis_index("s")
    base_e = s * TPT
    zero16 = jnp.zeros((16,), jnp.float32)
    iota16 = lax.iota(jnp.int32, 16)
    zidx = jnp.zeros((16,), jnp.int32)
    c128 = jnp.full((16,), 128, jnp.int32)
    unit = jnp.where(iota16 == 0, 1.0, 0.0).astype(jnp.float32)

    def _z1(i, _):
        for v in range(CW // 16):
            zb[i, pl.ds(v * 16, 16)] = zero16
        return 0
    lax.fori_loop(0, 16, _z1, 0)
    for k in range(NPT // 16):
        pltpu.sync_copy(zb, num_s.at[pl.ds(s * NPT + k * 16, 16)])
    plsc.subcore_barrier()

    def issueA(b, k):
        off = base_e + b * EB
        pltpu.async_copy(src_hbm.at[pl.ds(off, EB)], srcv[k], semA[k])
        pltpu.async_copy(dst_hbm.at[pl.ds(off, EB)], dstv[k], semA[k])

    def waitA(k):
        pltpu.make_async_copy(src_hbm.at[pl.ds(0, EB)], srcv[k], semA[k]).wait()
        pltpu.make_async_copy(dst_hbm.at[pl.ds(0, EB)], dstv[k], semA[k]).wait()

    def buildIdx(k):
        for v in range(EB // 16):
            sl = pl.ds(v * 16, 16)
            dv = dstv[k][sl]
            gidx[k][sl] = srcv[k][sl] + c * N
            didx[k][sl] = dv + c * NPAD
            sidx[k][sl] = dv

    def issueG(k):
        pltpu.async_copy(xwext_hbm.at[gidx[k]], rowsx[k], semG[k])
        pltpu.async_copy(adst_hbm.at[didx[k]], ddata[k], semG[k])

    def waitG(k):
        pltpu.make_async_copy(xwext_hbm.at[gidx[k]], rowsx[k], semG[k]).wait()
        pltpu.make_async_copy(adst_hbm.at[didx[k]], ddata[k], semG[k]).wait()

    def waitS(k):
        pltpu.make_async_copy(rowsx[k], num_s.at[sidx[k]], semS[k]).wait()

    def _p1(b, k):
        @pl.when(b + 1 < NBAT)
        def _():
            waitA(1 - k)

            @pl.when(b >= 1)
            def _():
                waitS(1 - k)
            buildIdx(1 - k)
            issueG(1 - k)

            @pl.when(b + 2 < NBAT)
            def _():
                issueA(b + 2, k)
        waitG(k)
        off = base_e + b * EB
        for v in range(EB // 16):
            sl = pl.ds(v * 16, 16)
            rid = v * 16 + iota16
            asv = plsc.load_gather(rowsx[k], [rid, c128])
            adv = plsc.load_gather(ddata[k], [rid, zidx])
            e = asv + adv
            e = jnp.where(e > 0, e, 0.2 * e)
            exv = jnp.exp(e)
            ge = off + v * 16 + iota16
            exv = jnp.where(ge < ESL, exv, 0.0)
            exb[k][sl] = exv
            ex_all[pl.ds(b * EB + v * 16, 16)] = exv

        def _scale(j, _):
            exj = exb[k][pl.ds(j, 16)][0]
            for v in range(8):
                sl = pl.ds(v * 16, 16)
                rowsx[k][j, sl] = rowsx[k][j, sl] * exj
            rowsx[k][j, pl.ds(128, 16)] = unit * exj
            return 0
        lax.fori_loop(0, EB, _scale, 0)
        pltpu.async_copy(rowsx[k], num_s.at[sidx[k]], semS[k], add=True)

    issueA(0, 0)
    waitA(0)
    buildIdx(0)
    issueG(0)
    issueA(1, 1)

    def _pair1(i, _):
        _p1(2 * i, 0)
        _p1(2 * i + 1, 1)
        return 0
    lax.fori_loop(0, NBAT // 2, _pair1, 0)
    waitS(0)
    waitS(1)
    plsc.subcore_barrier()

    pltpu.sync_copy(num_s.at[pl.ds(s * NPT, NPT)],
                    num_hbm.at[pl.ds(c * NPAD + s * NPT, NPT)])
    plsc.subcore_barrier()

    def issueA2(b, k):
        off = base_e + b * EB
        pltpu.async_copy(dst_hbm.at[pl.ds(off, EB)], dstv[k], semA[k])

    def waitA2(k):
        pltpu.make_async_copy(dst_hbm.at[pl.ds(0, EB)], dstv[k], semA[k]).wait()

    def buildIdx2(k):
        for v in range(EB // 16):
            sl = pl.ds(v * 16, 16)
            didx[k][sl] = dstv[k][sl] + c * NPAD

    def issueG2(k):
        pltpu.async_copy(num_hbm.at[didx[k]], rowsx[k], semG[k])

    def waitG2(k):
        pltpu.make_async_copy(num_hbm.at[didx[k]], rowsx[k], semG[k]).wait()

    def waitS2(b, k):
        off = base_e + b * EB
        pltpu.make_async_copy(
            alpha_b[k], alpha_hbm.at[c, pl.ds(off, EB)], semS[k]).wait()

    def _p2(b, k):
        @pl.when(b + 1 < NBAT)
        def _():
            waitA2(1 - k)
            buildIdx2(1 - k)
            issueG2(1 - k)

            @pl.when(b + 2 < NBAT)
            def _():
                issueA2(b + 2, k)
        waitG2(k)

        @pl.when(b >= 2)
        def _():
            waitS2(b - 2, k)
        off = base_e + b * EB
        for v in range(EB // 16):
            sl = pl.ds(v * 16, 16)
            rid = v * 16 + iota16
            dv = plsc.load_gather(rowsx[k], [rid, c128])
            exv = ex_all[pl.ds(b * EB + v * 16, 16)]
            alpha_b[k][sl] = exv / (dv + 1e-16)
        pltpu.async_copy(alpha_b[k], alpha_hbm.at[c, pl.ds(off, EB)], semS[k])

    issueA2(0, 0)
    waitA2(0)
    buildIdx2(0)
    issueG2(0)
    issueA2(1, 1)

    def _pair2(i, _):
        _p2(2 * i, 0)
        _p2(2 * i + 1, 1)
        return 0
    lax.fori_loop(0, NBAT // 2, _pair2, 0)
    waitS2(NBAT - 2, 0)
    waitS2(NBAT - 1, 1)


def _edge_phase(src, dst, xwext, adst_tab):
    mesh = plsc.VectorSubcoreMesh(core_axis_name="c", subcore_axis_name="s")
    dbl = lambda t: [t, t]
    f = pl.kernel(
        _edge_body,
        out_type=[
            jax.ShapeDtypeStruct((H * NPAD, CW), jnp.float32),
            jax.ShapeDtypeStruct((H, EPAD), jnp.float32),
        ],
        mesh=mesh,
        compiler_params=pltpu.CompilerParams(
            needs_layout_passes=False, use_tc_tiling_on_sc=False),
        scratch_types=(
            dbl(pltpu.VMEM((EB,), jnp.int32))
            + dbl(pltpu.VMEM((EB,), jnp.int32))
            + dbl(pltpu.VMEM((EB,), jnp.int32))
            + dbl(pltpu.VMEM((EB,), jnp.int32))
            + dbl(pltpu.VMEM((EB,), jnp.int32))
            + dbl(pltpu.VMEM((EB + 16,), jnp.float32))
            + dbl(pltpu.VMEM((EB, CW), jnp.float32))
            + dbl(pltpu.VMEM((EB, 16), jnp.float32))
            + [pltpu.VMEM((TPT,), jnp.float32)]
            + dbl(pltpu.VMEM((EB,), jnp.float32))
            + [pltpu.VMEM((16, CW), jnp.float32)]
            + [pltpu.SemaphoreType.DMA] * 6
            + [pltpu.VMEM_SHARED((NPAD, CW), jnp.float32)]
        ),
    )
    return f(src, dst, xwext, adst_tab)


def _mlp_kernel(num_ref, bc_ref, wa_ref, ba_ref, w1_ref, b1_ref,
                w2_ref, b2_ref, out_ref):
    arr = num_ref[...]
    den = arr[:, :, 128]
    den = jnp.where(den > 0, den, 1.0)
    h0 = jnp.concatenate(
        [arr[0, :, :C] / den[0][:, None], arr[1, :, :C] / den[1][:, None]],
        axis=1)
    h0 = jax.nn.relu(h0 + bc_ref[...])
    h1 = jax.nn.relu(jnp.dot(h0, wa_ref[...], preferred_element_type=jnp.float32)
                     + ba_ref[...])
    h2 = jax.nn.relu(jnp.dot(h1, w1_ref[...], preferred_element_type=jnp.float32)
                     + b1_ref[...])
    h3 = jnp.dot(h2, w2_ref[...], preferred_element_type=jnp.float32) + b2_ref[...]
    out_ref[...] = h3


def _mlp(num, b_conv, Wa, ba, W1, b1, W2, b2):
    BN = 1024
    grid = (NPAD // BN,)
    return pl.pallas_call(
        _mlp_kernel,
        grid=grid,
        in_specs=[
            pl.BlockSpec((2, BN, CW), lambda i: (0, i, 0)),
            pl.BlockSpec((1, H * C), lambda i: (0, 0)),
            pl.BlockSpec((H * C, 128), lambda i: (0, 0)),
            pl.BlockSpec((1, 128), lambda i: (0, 0)),
            pl.BlockSpec((128, 64), lambda i: (0, 0)),
            pl.BlockSpec((1, 64), lambda i: (0, 0)),
            pl.BlockSpec((64, 3), lambda i: (0, 0)),
            pl.BlockSpec((1, 3), lambda i: (0, 0)),
        ],
        out_specs=pl.BlockSpec((BN, 3), lambda i: (i, 0)),
        out_shape=jax.ShapeDtypeStruct((NPAD, 3), jnp.float32),
    )(num, b_conv.reshape(1, H * C), Wa, ba.reshape(1, 128),
      W1, b1.reshape(1, 64), W2, b2.reshape(1, 3))


def _cdist_kernel(hi_ref, hj_ref, out_ref):
    hi = hi_ref[...]
    hj = hj_ref[...]
    g = jax.lax.dot_general(hi, hj, (((1,), (1,)), ((), ())),
                            preferred_element_type=jnp.float32)
    sqi = jnp.sum(hi * hi, axis=1, keepdims=True)
    sqj = jnp.sum(hj * hj, axis=1)[None, :]
    d2 = sqi + sqj - 2.0 * g
    d2 = jnp.maximum(d2, 0.0)
    d = jnp.sqrt(jnp.where(d2 > 0, d2, 1.0))
    out_ref[...] = jnp.where(d2 > 0, d, 0.0)


def _cdist(h3):
    B = 512
    g = pl.cdiv(N, B)
    return pl.pallas_call(
        _cdist_kernel,
        grid=(g, g),
        in_specs=[
            pl.BlockSpec((B, 3), lambda i, j: (i, 0)),
            pl.BlockSpec((B, 3), lambda i, j: (j, 0)),
        ],
        out_specs=pl.BlockSpec((B, B), lambda i, j: (i, j)),
        out_shape=jax.ShapeDtypeStruct((N, N), jnp.float32),
    )(h3, h3)


def kernel(x, edge_index, W, att_src, att_dst, b_conv, Wa, ba, W1, b1, W2, b2):
    loop = jnp.arange(N, dtype=edge_index.dtype)
    src = jnp.concatenate([edge_index[0], loop])
    dst = jnp.concatenate([edge_index[1], loop])

    xwext, adst16 = _compute_features(x, W, att_src, att_dst)

    pad = jnp.zeros((EPAD - ESL,), src.dtype)
    src_p = jnp.concatenate([src, pad])
    dst_p = jnp.concatenate([dst, pad])
    adst_tab = adst16.reshape(H * NPAD, 16)
    numden, alpha_he = _edge_phase(src_p, dst_p, xwext, adst_tab)
    alpha = jnp.stack([alpha_he[0, :ESL], alpha_he[1, :ESL]], axis=1)

    num = numden.reshape(H, NPAD, CW)
    h3 = _mlp(num, b_conv, Wa, ba, W1, b1, W2, b2)
    dist = _cdist(h3)
    edge_index_sl = jnp.stack([src, dst], axis=0)
    return dist, (edge_index_sl, alpha)


# speedup vs baseline: 24.9831x; 1.2455x over previous
import functools

import jax
import jax.numpy as jnp
from jax import lax
from jax.experimental import pallas as pl
from jax.experimental.pallas import tpu as pltpu
from jax.experimental.pallas import tpu_sc as plsc

N = 10000
E = 160000
D = 256
H = 2
C = 128
ESL = E + N
NPAD = 10240
NTILES = 16
TPT = 10752
EPAD = NTILES * TPT
NB = TPT // 128
NPT = NPAD // NTILES


def _feat_kernel(x_ref, w_ref, asrc_ref, adst_ref, xwext_ref, adst16_ref):
    h = pl.program_id(0)
    bn = x_ref.shape[0]
    xwb = jnp.dot(x_ref[...], w_ref[...], preferred_element_type=jnp.float32)
    a_s = jax.lax.dot_general(xwb, asrc_ref[pl.ds(h, 1), :],
                              (((1,), (1,)), ((), ())),
                              preferred_element_type=jnp.float32)
    a_d = jax.lax.dot_general(xwb, adst_ref[pl.ds(h, 1), :],
                              (((1,), (1,)), ((), ())),
                              preferred_element_type=jnp.float32)
    zpad = jnp.zeros((bn, 15), jnp.float32)
    xwext_ref[...] = jnp.concatenate([xwb, a_s, zpad], axis=1)
    adst16_ref[...] = jnp.concatenate([a_d, zpad], axis=1)[None]


def _compute_features(x, W, att_src, att_dst):
    BN = 1000
    grid = (H, N // BN)
    nb = N // BN
    return pl.pallas_call(
        _feat_kernel,
        grid=grid,
        in_specs=[
            pl.BlockSpec((BN, D), lambda h, i: (i, 0)),
            pl.BlockSpec((D, C), lambda h, i: (0, h)),
            pl.BlockSpec((H, C), lambda h, i: (0, 0)),
            pl.BlockSpec((H, C), lambda h, i: (0, 0)),
        ],
        out_specs=[
            pl.BlockSpec((BN, CW), lambda h, i: (h * nb + i, 0)),
            pl.BlockSpec((1, BN, 16), lambda h, i: (h, i, 0)),
        ],
        out_shape=[
            jax.ShapeDtypeStruct((H * N, CW), jnp.float32),
            jax.ShapeDtypeStruct((H, NPAD, 16), jnp.float32),
        ],
    )(x, W, att_src, att_dst)


CW = 144
EB = 64
NBAT = TPT // EB


def _edge_body(src_hbm, dst_hbm, xwext_hbm, adst_hbm,
               num_hbm, alpha_hbm,
               srcv0, srcv1, dstv0, dstv1, gidx0, gidx1, didx0, didx1,
               sidx0, sidx1, exb0, exb1, rowsx0, rowsx1, ddata0, ddata1,
               ex_all, ab0, ab1, zb,
               semA0, semA1, semG0, semG1, semS0, semS1, num_s):
    srcv = (srcv0, srcv1)
    dstv = (dstv0, dstv1)
    gidx = (gidx0, gidx1)
    didx = (didx0, didx1)
    sidx = (sidx0, sidx1)
    exb = (exb0, exb1)
    rowsx = (rowsx0, rowsx1)
    ddata = (ddata0, ddata1)
    alpha_b = (ab0, ab1)
    semA = (semA0, semA1)
    semG = (semG0, semG1)
    semS = (semS0, semS1)

    c = lax.axis_index("c")
    s = lax.axis_index("s")
    base_e = s * TPT
    zero16 = jnp.zeros((16,), jnp.float32)
    iota16 = lax.iota(jnp.int32, 16)
    zidx = jnp.zeros((16,), jnp.int32)
    c128 = jnp.full((16,), 128, jnp.int32)
    unit = jnp.where(iota16 == 0, 1.0, 0.0).astype(jnp.float32)

    def _z1(i, _):
        for v in range(CW // 16):
            zb[i, pl.ds(v * 16, 16)] = zero16
        return 0
    lax.fori_loop(0, 16, _z1, 0)
    for k in range(NPT // 16):
        pltpu.sync_copy(zb, num_s.at[pl.ds(s * NPT + k * 16, 16)])
    plsc.subcore_barrier()

    def issueA(b, k):
        off = base_e + b * EB
        pltpu.async_copy(src_hbm.at[pl.ds(off, EB)], srcv[k], semA[k])
        pltpu.async_copy(dst_hbm.at[pl.ds(off, EB)], dstv[k], semA[k])

    def waitA(k):
        pltpu.make_async_copy(src_hbm.at[pl.ds(0, EB)], srcv[k], semA[k]).wait()
        pltpu.make_async_copy(dst_hbm.at[pl.ds(0, EB)], dstv[k], semA[k]).wait()

    def buildIdx(k):
        for v in range(EB // 16):
            sl = pl.ds(v * 16, 16)
            dv = dstv[k][sl]
            gidx[k][sl] = srcv[k][sl] + c * N
            didx[k][sl] = dv + c * NPAD
            sidx[k][sl] = dv

    def issueG(k):
        pltpu.async_copy(xwext_hbm.at[gidx[k]], rowsx[k], semG[k])
        pltpu.async_copy(adst_hbm.at[didx[k]], ddata[k], semG[k])

    def waitG(k):
        pltpu.make_async_copy(xwext_hbm.at[gidx[k]], rowsx[k], semG[k]).wait()
        pltpu.make_async_copy(adst_hbm.at[didx[k]], ddata[k], semG[k]).wait()

    def waitS(k):
        pltpu.make_async_copy(rowsx[k], num_s.at[sidx[k]], semS[k]).wait()

    def _p1(b, k):
        @pl.when(b + 1 < NBAT)
        def _():
            waitA(1 - k)

            @pl.when(b >= 1)
            def _():
                waitS(1 - k)
            buildIdx(1 - k)
            issueG(1 - k)

            @pl.when(b + 2 < NBAT)
            def _():
                issueA(b + 2, k)
        waitG(k)
        off = base_e + b * EB
        for v in range(EB // 16):
            sl = pl.ds(v * 16, 16)
            rid = v * 16 + iota16
            asv = plsc.load_gather(rowsx[k], [rid, c128])
            adv = plsc.load_gather(ddata[k], [rid, zidx])
            e = asv + adv
            e = jnp.where(e > 0, e, 0.2 * e)
            exv = jnp.exp(e)
            ge = off + v * 16 + iota16
            exv = jnp.where(ge < ESL, exv, 0.0)
            exb[k][sl] = exv
            ex_all[pl.ds(b * EB + v * 16, 16)] = exv

        def _scale(j, _):
            exj = exb[k][pl.ds(j, 16)][0]
            for v in range(8):
                sl = pl.ds(v * 16, 16)
                rowsx[k][j, sl] = rowsx[k][j, sl] * exj
            rowsx[k][j, pl.ds(128, 16)] = unit * exj
            return 0
        lax.fori_loop(0, EB, _scale, 0)
        pltpu.async_copy(rowsx[k], num_s.at[sidx[k]], semS[k], add=True)

    issueA(0, 0)
    waitA(0)
    buildIdx(0)
    issueG(0)
    issueA(1, 1)

    def _pair1(i, _):
        _p1(2 * i, 0)
        _p1(2 * i + 1, 1)
        return 0
    lax.fori_loop(0, NBAT // 2, _pair1, 0)
    waitS(0)
    waitS(1)
    plsc.subcore_barrier()

    pltpu.sync_copy(num_s.at[pl.ds(s * NPT, NPT)],
                    num_hbm.at[pl.ds(c * NPAD + s * NPT, NPT)])
    plsc.subcore_barrier()

    def issueA2(b, k):
        off = base_e + b * EB
        pltpu.async_copy(dst_hbm.at[pl.ds(off, EB)], dstv[k], semA[k])

    def waitA2(k):
        pltpu.make_async_copy(dst_hbm.at[pl.ds(0, EB)], dstv[k], semA[k]).wait()

    def buildIdx2(k):
        for v in range(EB // 16):
            sl = pl.ds(v * 16, 16)
            didx[k][sl] = dstv[k][sl] + c * NPAD

    def issueG2(k):
        pltpu.async_copy(num_hbm.at[didx[k]], rowsx[k], semG[k])

    def waitG2(k):
        pltpu.make_async_copy(num_hbm.at[didx[k]], rowsx[k], semG[k]).wait()

    def waitS2(b, k):
        off = base_e + b * EB
        pltpu.make_async_copy(
            alpha_b[k], alpha_hbm.at[c, pl.ds(off, EB)], semS[k]).wait()

    def _p2(b, k):
        @pl.when(b + 1 < NBAT)
        def _():
            waitA2(1 - k)
            buildIdx2(1 - k)
            issueG2(1 - k)

            @pl.when(b + 2 < NBAT)
            def _():
                issueA2(b + 2, k)
        waitG2(k)

        @pl.when(b >= 2)
        def _():
            waitS2(b - 2, k)
        off = base_e + b * EB
        for v in range(EB // 16):
            sl = pl.ds(v * 16, 16)
            rid = v * 16 + iota16
            dv = plsc.load_gather(rowsx[k], [rid, c128])
            exv = ex_all[pl.ds(b * EB + v * 16, 16)]
            alpha_b[k][sl] = exv / (dv + 1e-16)
        pltpu.async_copy(alpha_b[k], alpha_hbm.at[c, pl.ds(off, EB)], semS[k])

    issueA2(0, 0)
    waitA2(0)
    buildIdx2(0)
    issueG2(0)
    issueA2(1, 1)

    def _pair2(i, _):
        _p2(2 * i, 0)
        _p2(2 * i + 1, 1)
        return 0
    lax.fori_loop(0, NBAT // 2, _pair2, 0)
    waitS2(NBAT - 2, 0)
    waitS2(NBAT - 1, 1)


def _edge_phase(src, dst, xwext, adst_tab):
    mesh = plsc.VectorSubcoreMesh(core_axis_name="c", subcore_axis_name="s")
    dbl = lambda t: [t, t]
    f = pl.kernel(
        _edge_body,
        out_type=[
            jax.ShapeDtypeStruct((H * NPAD, CW), jnp.float32),
            jax.ShapeDtypeStruct((H, EPAD), jnp.float32),
        ],
        mesh=mesh,
        compiler_params=pltpu.CompilerParams(
            needs_layout_passes=False, use_tc_tiling_on_sc=False),
        scratch_types=(
            dbl(pltpu.VMEM((EB,), jnp.int32))
            + dbl(pltpu.VMEM((EB,), jnp.int32))
            + dbl(pltpu.VMEM((EB,), jnp.int32))
            + dbl(pltpu.VMEM((EB,), jnp.int32))
            + dbl(pltpu.VMEM((EB,), jnp.int32))
            + dbl(pltpu.VMEM((EB + 16,), jnp.float32))
            + dbl(pltpu.VMEM((EB, CW), jnp.float32))
            + dbl(pltpu.VMEM((EB, 16), jnp.float32))
            + [pltpu.VMEM((TPT,), jnp.float32)]
            + dbl(pltpu.VMEM((EB,), jnp.float32))
            + [pltpu.VMEM((16, CW), jnp.float32)]
            + [pltpu.SemaphoreType.DMA] * 6
            + [pltpu.VMEM_SHARED((NPAD, CW), jnp.float32)]
        ),
    )
    return f(src, dst, xwext, adst_tab)


def _mlp_kernel(num_ref, bc_ref, wa_ref, ba_ref, w1_ref, b1_ref,
                w2_ref, b2_ref, out_ref):
    arr = num_ref[...]
    den = arr[:, :, 128]
    den = jnp.where(den > 0, den, 1.0)
    h0 = jnp.concatenate(
        [arr[0, :, :C] / den[0][:, None], arr[1, :, :C] / den[1][:, None]],
        axis=1)
    h0 = jax.nn.relu(h0 + bc_ref[...])
    h1 = jax.nn.relu(jnp.dot(h0, wa_ref[...], preferred_element_type=jnp.float32)
                     + ba_ref[...])
    h2 = jax.nn.relu(jnp.dot(h1, w1_ref[...], preferred_element_type=jnp.float32)
                     + b1_ref[...])
    h3 = jnp.dot(h2, w2_ref[...], preferred_element_type=jnp.float32) + b2_ref[...]
    out_ref[...] = h3


def _mlp(num, b_conv, Wa, ba, W1, b1, W2, b2):
    BN = 1024
    grid = (NPAD // BN,)
    return pl.pallas_call(
        _mlp_kernel,
        grid=grid,
        in_specs=[
            pl.BlockSpec((2, BN, CW), lambda i: (0, i, 0)),
            pl.BlockSpec((1, H * C), lambda i: (0, 0)),
            pl.BlockSpec((H * C, 128), lambda i: (0, 0)),
            pl.BlockSpec((1, 128), lambda i: (0, 0)),
            pl.BlockSpec((128, 64), lambda i: (0, 0)),
            pl.BlockSpec((1, 64), lambda i: (0, 0)),
            pl.BlockSpec((64, 3), lambda i: (0, 0)),
            pl.BlockSpec((1, 3), lambda i: (0, 0)),
        ],
        out_specs=pl.BlockSpec((BN, 3), lambda i: (i, 0)),
        out_shape=jax.ShapeDtypeStruct((NPAD, 3), jnp.float32),
    )(num, b_conv.reshape(1, H * C), Wa, ba.reshape(1, 128),
      W1, b1.reshape(1, 64), W2, b2.reshape(1, 3))


def _cdist_kernel(hi_ref, hj_ref, out_ref):
    hi = hi_ref[...]
    hj = hj_ref[...]
    g = jax.lax.dot_general(hi, hj, (((1,), (1,)), ((), ())),
                            preferred_element_type=jnp.float32)
    sqi = jnp.sum(hi * hi, axis=1, keepdims=True)
    sqj = jnp.sum(hj * hj, axis=1)[None, :]
    d2 = sqi + sqj - 2.0 * g
    out_ref[...] = jnp.sqrt(jnp.maximum(d2, 0.0))


def _cdist(h3):
    BI, BJ = 1024, 1024
    return pl.pallas_call(
        _cdist_kernel,
        grid=(pl.cdiv(N, BI), pl.cdiv(N, BJ)),
        in_specs=[
            pl.BlockSpec((BI, 3), lambda i, j: (i, 0)),
            pl.BlockSpec((BJ, 3), lambda i, j: (j, 0)),
        ],
        out_specs=pl.BlockSpec((BI, BJ), lambda i, j: (i, j)),
        out_shape=jax.ShapeDtypeStruct((N, N), jnp.float32),
    )(h3, h3)


def kernel(x, edge_index, W, att_src, att_dst, b_conv, Wa, ba, W1, b1, W2, b2):
    loop = jnp.arange(N, dtype=edge_index.dtype)
    src = jnp.concatenate([edge_index[0], loop])
    dst = jnp.concatenate([edge_index[1], loop])

    xwext, adst16 = _compute_features(x, W, att_src, att_dst)

    pad = jnp.zeros((EPAD - ESL,), src.dtype)
    src_p = jnp.concatenate([src, pad])
    dst_p = jnp.concatenate([dst, pad])
    adst_tab = adst16.reshape(H * NPAD, 16)
    numden, alpha_he = _edge_phase(src_p, dst_p, xwext, adst_tab)
    alpha = jnp.stack([alpha_he[0, :ESL], alpha_he[1, :ESL]], axis=1)

    num = numden.reshape(H, NPAD, CW)
    h3 = _mlp(num, b_conv, Wa, ba, W1, b1, W2, b2)
    dist = _cdist(h3)
    edge_index_sl = jnp.stack([src, dst], axis=0)
    return dist, (edge_index_sl, alpha)


# speedup vs baseline: 29.1110x; 1.1652x over previous
import functools

import jax
import jax.numpy as jnp
from jax import lax
from jax.experimental import pallas as pl
from jax.experimental.pallas import tpu as pltpu
from jax.experimental.pallas import tpu_sc as plsc

N = 10000
E = 160000
D = 256
H = 2
C = 128
ESL = E + N
NPAD = 10240
NTILES = 16
TPT = 10752
EPAD = NTILES * TPT
NB = TPT // 128
NPT = NPAD // NTILES


def _feat_kernel(x_ref, w_ref, asrc_ref, adst_ref, xwext_ref, adst16_ref):
    h = pl.program_id(0)
    bn = x_ref.shape[0]
    xwb = jnp.dot(x_ref[...], w_ref[...], preferred_element_type=jnp.float32)
    a_s = jax.lax.dot_general(xwb, asrc_ref[pl.ds(h, 1), :],
                              (((1,), (1,)), ((), ())),
                              preferred_element_type=jnp.float32)
    a_d = jax.lax.dot_general(xwb, adst_ref[pl.ds(h, 1), :],
                              (((1,), (1,)), ((), ())),
                              preferred_element_type=jnp.float32)
    zpad = jnp.zeros((bn, 15), jnp.float32)
    xwext_ref[...] = jnp.concatenate([xwb, a_s, zpad], axis=1)
    adst16_ref[...] = jnp.concatenate([a_d, zpad], axis=1)[None]


def _compute_features(x, W, att_src, att_dst):
    BN = 1000
    grid = (H, N // BN)
    nb = N // BN
    return pl.pallas_call(
        _feat_kernel,
        grid=grid,
        in_specs=[
            pl.BlockSpec((BN, D), lambda h, i: (i, 0)),
            pl.BlockSpec((D, C), lambda h, i: (0, h)),
            pl.BlockSpec((H, C), lambda h, i: (0, 0)),
            pl.BlockSpec((H, C), lambda h, i: (0, 0)),
        ],
        out_specs=[
            pl.BlockSpec((BN, CW), lambda h, i: (h * nb + i, 0)),
            pl.BlockSpec((1, BN, 16), lambda h, i: (h, i, 0)),
        ],
        out_shape=[
            jax.ShapeDtypeStruct((H * N, CW), jnp.float32),
            jax.ShapeDtypeStruct((H, NPAD, 16), jnp.float32),
        ],
    )(x, W, att_src, att_dst)


CW = 144
EB = 64
NBAT = TPT // EB


def _edge_body(src_hbm, dst_hbm, xwext_hbm, adst_hbm,
               num_hbm, alpha_hbm, den2_hbm,
               srcv0, srcv1, dstv0, dstv1, gidx0, gidx1, didx0, didx1,
               sidx0, sidx1, exb0, exb1, rowsx0, rowsx1, ddata0, ddata1,
               ex_all, ab0, ab1, zb,
               semA0, semA1, semG0, semG1, semS0, semS1, num_s):
    srcv = (srcv0, srcv1)
    dstv = (dstv0, dstv1)
    gidx = (gidx0, gidx1)
    didx = (didx0, didx1)
    sidx = (sidx0, sidx1)
    exb = (exb0, exb1)
    rowsx = (rowsx0, rowsx1)
    ddata = (ddata0, ddata1)
    alpha_b = (ab0, ab1)
    semA = (semA0, semA1)
    semG = (semG0, semG1)
    semS = (semS0, semS1)

    c = lax.axis_index("c")
    s = lax.axis_index("s")
    base_e = s * TPT
    zero16 = jnp.zeros((16,), jnp.float32)
    iota16 = lax.iota(jnp.int32, 16)
    zidx = jnp.zeros((16,), jnp.int32)
    c128 = jnp.full((16,), 128, jnp.int32)
    unit = jnp.where(iota16 == 0, 1.0, 0.0).astype(jnp.float32)

    def _z1(i, _):
        for v in range(CW // 16):
            zb[i, pl.ds(v * 16, 16)] = zero16
        return 0
    lax.fori_loop(0, 16, _z1, 0)
    for k in range(NPT // 16):
        pltpu.sync_copy(zb, num_s.at[pl.ds(s * NPT + k * 16, 16)])
    plsc.subcore_barrier()

    def issueA(b, k):
        off = base_e + b * EB
        pltpu.async_copy(src_hbm.at[pl.ds(off, EB)], srcv[k], semA[k])
        pltpu.async_copy(dst_hbm.at[pl.ds(off, EB)], dstv[k], semA[k])

    def waitA(k):
        pltpu.make_async_copy(src_hbm.at[pl.ds(0, EB)], srcv[k], semA[k]).wait()
        pltpu.make_async_copy(dst_hbm.at[pl.ds(0, EB)], dstv[k], semA[k]).wait()

    def buildIdx(k):
        for v in range(EB // 16):
            sl = pl.ds(v * 16, 16)
            dv = dstv[k][sl]
            gidx[k][sl] = srcv[k][sl] + c * N
            didx[k][sl] = dv + c * NPAD
            sidx[k][sl] = dv

    def issueG(k):
        pltpu.async_copy(xwext_hbm.at[gidx[k]], rowsx[k], semG[k])
        pltpu.async_copy(adst_hbm.at[didx[k]], ddata[k], semG[k])

    def waitG(k):
        pltpu.make_async_copy(xwext_hbm.at[gidx[k]], rowsx[k], semG[k]).wait()
        pltpu.make_async_copy(adst_hbm.at[didx[k]], ddata[k], semG[k]).wait()

    def waitS(k):
        pltpu.make_async_copy(rowsx[k], num_s.at[sidx[k]], semS[k]).wait()

    def _p1(b, k):
        @pl.when(b + 1 < NBAT)
        def _():
            waitA(1 - k)

            @pl.when(b >= 1)
            def _():
                waitS(1 - k)
            buildIdx(1 - k)
            issueG(1 - k)

            @pl.when(b + 2 < NBAT)
            def _():
                issueA(b + 2, k)
        waitG(k)
        off = base_e + b * EB
        for v in range(EB // 16):
            sl = pl.ds(v * 16, 16)
            rid = v * 16 + iota16
            asv = plsc.load_gather(rowsx[k], [rid, c128])
            adv = plsc.load_gather(ddata[k], [rid, zidx])
            e = asv + adv
            e = jnp.where(e > 0, e, 0.2 * e)
            exv = jnp.exp(e)
            ge = off + v * 16 + iota16
            exv = jnp.where(ge < ESL, exv, 0.0)
            exb[k][sl] = exv
            ex_all[pl.ds(b * EB + v * 16, 16)] = exv

        def _scale(j, _):
            exj = exb[k][pl.ds(j, 16)][0]
            for v in range(8):
                sl = pl.ds(v * 16, 16)
                rowsx[k][j, sl] = rowsx[k][j, sl] * exj
            rowsx[k][j, pl.ds(128, 16)] = unit * exj
            return 0
        lax.fori_loop(0, EB, _scale, 0)
        pltpu.async_copy(rowsx[k], num_s.at[sidx[k]], semS[k], add=True)

    issueA(0, 0)
    waitA(0)
    buildIdx(0)
    issueG(0)
    issueA(1, 1)

    def _pair1(i, _):
        _p1(2 * i, 0)
        _p1(2 * i + 1, 1)
        return 0
    lax.fori_loop(0, NBAT // 2, _pair1, 0)
    waitS(0)
    waitS(1)
    plsc.subcore_barrier()

    pltpu.sync_copy(num_s.at[pl.ds(s * NPT, NPT)],
                    num_hbm.at[pl.ds(c * NPAD + s * NPT, NPT)])

    def _ext(j, _):
        pltpu.sync_copy(num_s.at[pl.ds(s * NPT + j * 16, 16)], zb)
        denv = plsc.load_gather(zb, [iota16, c128])
        ddata0[j, pl.ds(0, 16)] = denv
        return 0
    lax.fori_loop(0, NPT // 16, _ext, 0)
    pltpu.sync_copy(ddata0.at[pl.ds(0, NPT // 16)],
                    den2_hbm.at[pl.ds(c * (NPAD // 16) + s * (NPT // 16),
                                      NPT // 16)])
    plsc.subcore_barrier()

    def issueA2(b, k):
        off = base_e + b * EB
        pltpu.async_copy(dst_hbm.at[pl.ds(off, EB)], dstv[k], semA[k])

    def waitA2(k):
        pltpu.make_async_copy(dst_hbm.at[pl.ds(0, EB)], dstv[k], semA[k]).wait()

    def buildIdx2(k):
        for v in range(EB // 16):
            sl = pl.ds(v * 16, 16)
            dv16 = dstv[k][sl]
            didx[k][sl] = jnp.right_shift(dv16, 4) + c * (NPAD // 16)
            sidx[k][sl] = dv16 & 15

    def issueG2(k):
        pltpu.async_copy(den2_hbm.at[didx[k]], ddata[k], semG[k])

    def waitG2(k):
        pltpu.make_async_copy(den2_hbm.at[didx[k]], ddata[k], semG[k]).wait()

    def waitS2(b, k):
        off = base_e + b * EB
        pltpu.make_async_copy(
            alpha_b[k], alpha_hbm.at[c, pl.ds(off, EB)], semS[k]).wait()

    def _p2(b, k):
        @pl.when(b + 1 < NBAT)
        def _():
            waitA2(1 - k)
            buildIdx2(1 - k)
            issueG2(1 - k)

            @pl.when(b + 2 < NBAT)
            def _():
                issueA2(b + 2, k)
        waitG2(k)

        @pl.when(b >= 2)
        def _():
            waitS2(b - 2, k)
        off = base_e + b * EB
        for v in range(EB // 16):
            sl = pl.ds(v * 16, 16)
            rid = v * 16 + iota16
            dv = plsc.load_gather(ddata[k], [rid, sidx[k][sl]])
            exv = ex_all[pl.ds(b * EB + v * 16, 16)]
            alpha_b[k][sl] = exv / (dv + 1e-16)
        pltpu.async_copy(alpha_b[k], alpha_hbm.at[c, pl.ds(off, EB)], semS[k])

    issueA2(0, 0)
    waitA2(0)
    buildIdx2(0)
    issueG2(0)
    issueA2(1, 1)

    def _pair2(i, _):
        _p2(2 * i, 0)
        _p2(2 * i + 1, 1)
        return 0
    lax.fori_loop(0, NBAT // 2, _pair2, 0)
    waitS2(NBAT - 2, 0)
    waitS2(NBAT - 1, 1)


def _edge_phase(src, dst, xwext, adst_tab):
    mesh = plsc.VectorSubcoreMesh(core_axis_name="c", subcore_axis_name="s")
    dbl = lambda t: [t, t]
    f = pl.kernel(
        _edge_body,
        out_type=[
            jax.ShapeDtypeStruct((H * NPAD, CW), jnp.float32),
            jax.ShapeDtypeStruct((H, EPAD), jnp.float32),
            jax.ShapeDtypeStruct((H * (NPAD // 16), 16), jnp.float32),
        ],
        mesh=mesh,
        compiler_params=pltpu.CompilerParams(
            needs_layout_passes=False, use_tc_tiling_on_sc=False),
        scratch_types=(
            dbl(pltpu.VMEM((EB,), jnp.int32))
            + dbl(pltpu.VMEM((EB,), jnp.int32))
            + dbl(pltpu.VMEM((EB,), jnp.int32))
            + dbl(pltpu.VMEM((EB,), jnp.int32))
            + dbl(pltpu.VMEM((EB,), jnp.int32))
            + dbl(pltpu.VMEM((EB + 16,), jnp.float32))
            + dbl(pltpu.VMEM((EB, CW), jnp.float32))
            + dbl(pltpu.VMEM((EB, 16), jnp.float32))
            + [pltpu.VMEM((TPT,), jnp.float32)]
            + dbl(pltpu.VMEM((EB,), jnp.float32))
            + [pltpu.VMEM((16, CW), jnp.float32)]
            + [pltpu.SemaphoreType.DMA] * 6
            + [pltpu.VMEM_SHARED((NPAD, CW), jnp.float32)]
        ),
    )
    return f(src, dst, xwext, adst_tab)


def _mlp_kernel(num_ref, bc_ref, wa_ref, ba_ref, w1_ref, b1_ref,
                w2_ref, b2_ref, out_ref):
    arr = num_ref[...]
    den = arr[:, :, 128]
    den = jnp.where(den > 0, den, 1.0)
    h0 = jnp.concatenate(
        [arr[0, :, :C] / den[0][:, None], arr[1, :, :C] / den[1][:, None]],
        axis=1)
    h0 = jax.nn.relu(h0 + bc_ref[...])
    h1 = jax.nn.relu(jnp.dot(h0, wa_ref[...], preferred_element_type=jnp.float32)
                     + ba_ref[...])
    h2 = jax.nn.relu(jnp.dot(h1, w1_ref[...], preferred_element_type=jnp.float32)
                     + b1_ref[...])
    h3 = jnp.dot(h2, w2_ref[...], preferred_element_type=jnp.float32) + b2_ref[...]
    out_ref[...] = h3


def _mlp(num, b_conv, Wa, ba, W1, b1, W2, b2):
    BN = 1024
    grid = (NPAD // BN,)
    return pl.pallas_call(
        _mlp_kernel,
        grid=grid,
        in_specs=[
            pl.BlockSpec((2, BN, CW), lambda i: (0, i, 0)),
            pl.BlockSpec((1, H * C), lambda i: (0, 0)),
            pl.BlockSpec((H * C, 128), lambda i: (0, 0)),
            pl.BlockSpec((1, 128), lambda i: (0, 0)),
            pl.BlockSpec((128, 64), lambda i: (0, 0)),
            pl.BlockSpec((1, 64), lambda i: (0, 0)),
            pl.BlockSpec((64, 3), lambda i: (0, 0)),
            pl.BlockSpec((1, 3), lambda i: (0, 0)),
        ],
        out_specs=pl.BlockSpec((BN, 3), lambda i: (i, 0)),
        out_shape=jax.ShapeDtypeStruct((NPAD, 3), jnp.float32),
    )(num, b_conv.reshape(1, H * C), Wa, ba.reshape(1, 128),
      W1, b1.reshape(1, 64), W2, b2.reshape(1, 3))


def _cdist_kernel(hi_ref, hj_ref, out_ref):
    hi = hi_ref[...]
    hj = hj_ref[...]
    g = jax.lax.dot_general(hi, hj, (((1,), (1,)), ((), ())),
                            preferred_element_type=jnp.float32)
    sqi = jnp.sum(hi * hi, axis=1, keepdims=True)
    sqj = jnp.sum(hj * hj, axis=1)[None, :]
    d2 = sqi + sqj - 2.0 * g
    out_ref[...] = jnp.sqrt(jnp.maximum(d2, 0.0))


def _cdist(h3):
    BI, BJ = 1024, 1024
    return pl.pallas_call(
        _cdist_kernel,
        grid=(pl.cdiv(N, BI), pl.cdiv(N, BJ)),
        in_specs=[
            pl.BlockSpec((BI, 3), lambda i, j: (i, 0)),
            pl.BlockSpec((BJ, 3), lambda i, j: (j, 0)),
        ],
        out_specs=pl.BlockSpec((BI, BJ), lambda i, j: (i, j)),
        out_shape=jax.ShapeDtypeStruct((N, N), jnp.float32),
    )(h3, h3)


def kernel(x, edge_index, W, att_src, att_dst, b_conv, Wa, ba, W1, b1, W2, b2):
    loop = jnp.arange(N, dtype=edge_index.dtype)
    src = jnp.concatenate([edge_index[0], loop])
    dst = jnp.concatenate([edge_index[1], loop])

    xwext, adst16 = _compute_features(x, W, att_src, att_dst)

    pad = jnp.zeros((EPAD - ESL,), src.dtype)
    src_p = jnp.concatenate([src, pad])
    dst_p = jnp.concatenate([dst, pad])
    adst_tab = adst16.reshape(H * NPAD, 16)
    numden, alpha_he, _den2 = _edge_phase(src_p, dst_p, xwext, adst_tab)
    alpha = jnp.stack([alpha_he[0, :ESL], alpha_he[1, :ESL]], axis=1)

    num = numden.reshape(H, NPAD, CW)
    h3 = _mlp(num, b_conv, Wa, ba, W1, b1, W2, b2)
    dist = _cdist(h3)
    edge_index_sl = jnp.stack([src, dst], axis=0)
    return dist, (edge_index_sl, alpha)
